# baseline (speedup 1.0000x reference)
import functools

import jax
import jax.numpy as jnp
from jax.experimental import pallas as pl
from jax.experimental.pallas import tpu as pltpu
from jax.experimental.pallas import tpu_sc as plsc

EPS = 1e-5
TILE = 1024
HCHUNK = 128
SC_NW = 32
SC_CH = 32


def _sc_gather(table, idx):
    n_idx = idx.shape[0]
    d = table.shape[1]
    b_per_w = n_idx // SC_NW
    assert n_idx % (8 * SC_NW) == 0
    assert b_per_w % SC_CH == 0
    mesh = plsc.VectorSubcoreMesh(core_axis_name="c", subcore_axis_name="s")

    @functools.partial(
        pl.kernel,
        mesh=mesh,
        out_type=jax.ShapeDtypeStruct((n_idx, d), table.dtype),
        scratch_types=[
            pltpu.VMEM((SC_CH,), jnp.int32),
            pltpu.VMEM((SC_CH, d), table.dtype),
            pltpu.SemaphoreType.DMA,
        ],
    )
    def gk(table_hbm, idx_hbm, out_hbm, idx_v, rows_v, sem):
        wid = jax.lax.axis_index("s") * 2 + jax.lax.axis_index("c")
        base = wid * b_per_w

        @pl.loop(0, b_per_w // SC_CH)
        def _(c):
            off = base + c * SC_CH
            pltpu.sync_copy(idx_hbm.at[pl.ds(off, SC_CH)], idx_v)
            pltpu.async_copy(table_hbm.at[idx_v], rows_v, sem).wait()
            pltpu.sync_copy(rows_v, out_hbm.at[pl.ds(off, SC_CH)])

    return gk(table, idx)


def _ffn_body(texp_ref, x_ref, w10, w30, w20, w11, w31, w21, out_ref):
    t = pl.program_id(0)
    h = pl.program_id(1)
    nh = pl.num_programs(1)
    e = texp_ref[t]
    xb = x_ref[...].astype(jnp.bfloat16)
    w1 = jnp.where(e == 0, w10[...], w11[...]).astype(jnp.bfloat16)
    w3 = jnp.where(e == 0, w30[...], w31[...]).astype(jnp.bfloat16)
    w2 = jnp.where(e == 0, w20[...], w21[...]).astype(jnp.bfloat16)
    dn = (((1,), (1,)), ((), ()))
    h1 = jax.lax.dot_general(xb, w1, dn, preferred_element_type=jnp.float32)
    h3 = jax.lax.dot_general(xb, w3, dn, preferred_element_type=jnp.float32)
    g = (h1 * jax.nn.sigmoid(h1) * h3).astype(jnp.bfloat16)
    contrib = jax.lax.dot_general(g, w2, dn, preferred_element_type=jnp.float32)

    @pl.when(h == 0)
    def _():
        out_ref[...] = contrib

    @pl.when(h > 0)
    def _():
        out_ref[...] += contrib

    @pl.when(h == nh - 1)
    def _():
        r = out_ref[...]
        out_ref[...] = r * jax.lax.rsqrt(
            jnp.mean(r * r, axis=-1, keepdims=True) + EPS)


def _ffn_sorted(xs, texp, w1_0, w3_0, w2_0, w1_1, w3_1, w2_1):
    nps, d = xs.shape
    hidden = w1_0.shape[0]
    nt = nps // TILE
    hc = hidden // HCHUNK

    def _w1map(expert):
        return lambda t, h, s: (jnp.where(s[t] == expert, h, 0), 0)

    def _w2map(expert):
        return lambda t, h, s: (0, jnp.where(s[t] == expert, h, 0))

    grid_spec = pltpu.PrefetchScalarGridSpec(
        num_scalar_prefetch=1,
        grid=(nt, hc),
        in_specs=[
            pl.BlockSpec((TILE, d), lambda t, h, s: (t, 0)),
            pl.BlockSpec((HCHUNK, d), _w1map(0)),
            pl.BlockSpec((HCHUNK, d), _w1map(0)),
            pl.BlockSpec((d, HCHUNK), _w2map(0)),
            pl.BlockSpec((HCHUNK, d), _w1map(1)),
            pl.BlockSpec((HCHUNK, d), _w1map(1)),
            pl.BlockSpec((d, HCHUNK), _w2map(1)),
        ],
        out_specs=pl.BlockSpec((TILE, d), lambda t, h, s: (t, 0)),
    )
    return pl.pallas_call(
        _ffn_body,
        grid_spec=grid_spec,
        out_shape=jax.ShapeDtypeStruct((nps, d), jnp.float32),
        compiler_params=pltpu.CompilerParams(
            dimension_semantics=("parallel", "arbitrary")),
    )(texp, xs, w1_0, w3_0, w2_0, w1_1, w3_1, w2_1)


def kernel(x, modality_masks, w1_0, w3_0, w2_0, w1_1, w3_1, w2_1):
    b, s, d = x.shape
    n = b * s
    nps = n + 2 * TILE

    ids = modality_masks[1].reshape(n).astype(jnp.int32)
    c1 = jnp.cumsum(ids)
    n0 = n - c1[-1]
    n0p = ((n0 + TILE - 1) // TILE) * TILE
    iot = jnp.arange(n, dtype=jnp.int32)
    dst = jnp.where(ids == 0, iot - c1, n0p + c1 - 1).astype(jnp.int32)
    src = jnp.zeros((nps,), jnp.int32).at[dst].set(iot)
    texp = (jnp.arange(nps // TILE, dtype=jnp.int32) >= (n0p // TILE)
            ).astype(jnp.int32)

    xf = x.reshape(n, d)
    xs = _sc_gather(xf, src)
    ys = _ffn_sorted(xs, texp, w1_0, w3_0, w2_0,
                     w1_1, w3_1, w2_1)
    out = _sc_gather(ys, dst)
    return out.reshape(b, s, d)


# speedup vs baseline: 1.6978x; 1.6978x over previous
import functools

import jax
import jax.numpy as jnp
from jax.experimental import pallas as pl
from jax.experimental.pallas import tpu as pltpu
from jax.experimental.pallas import tpu_sc as plsc

EPS = 1e-5
TILE = 1024
HCHUNK = 256
SC_NW = 32
SC_CH = 32


def _sc_gather(table, idx):
    n_idx = idx.shape[0]
    d = table.shape[1]
    b_per_w = n_idx // SC_NW
    assert n_idx % (8 * SC_NW) == 0
    assert b_per_w % SC_CH == 0
    mesh = plsc.VectorSubcoreMesh(core_axis_name="c", subcore_axis_name="s")

    @functools.partial(
        pl.kernel,
        mesh=mesh,
        out_type=jax.ShapeDtypeStruct((n_idx, d), table.dtype),
        scratch_types=[
            pltpu.VMEM((SC_CH,), jnp.int32),
            pltpu.VMEM((SC_CH, d), table.dtype),
            pltpu.SemaphoreType.DMA,
        ],
    )
    def gk(table_hbm, idx_hbm, out_hbm, idx_v, rows_v, sem):
        wid = jax.lax.axis_index("s") * 2 + jax.lax.axis_index("c")
        base = wid * b_per_w

        @pl.loop(0, b_per_w // SC_CH)
        def _(c):
            off = base + c * SC_CH
            pltpu.sync_copy(idx_hbm.at[pl.ds(off, SC_CH)], idx_v)
            pltpu.async_copy(table_hbm.at[idx_v], rows_v, sem).wait()
            pltpu.sync_copy(rows_v, out_hbm.at[pl.ds(off, SC_CH)])

    return gk(table, idx)


def _ffn_body(texp_ref, x_ref, w10, w30, w20, w11, w31, w21, out_ref):
    t = pl.program_id(0)
    h = pl.program_id(1)
    nh = pl.num_programs(1)
    e = texp_ref[t]
    xb = x_ref[...].astype(jnp.bfloat16)
    w1 = jnp.where(e == 0, w10[...], w11[...])
    w3 = jnp.where(e == 0, w30[...], w31[...])
    w2 = jnp.where(e == 0, w20[...], w21[...])
    dn = (((1,), (1,)), ((), ()))
    h1 = jax.lax.dot_general(xb, w1, dn, preferred_element_type=jnp.float32)
    h3 = jax.lax.dot_general(xb, w3, dn, preferred_element_type=jnp.float32)
    g = (h1 * jax.nn.sigmoid(h1) * h3).astype(jnp.bfloat16)
    contrib = jax.lax.dot_general(g, w2, dn, preferred_element_type=jnp.float32)

    @pl.when(h == 0)
    def _():
        out_ref[...] = contrib

    @pl.when(h > 0)
    def _():
        out_ref[...] += contrib

    @pl.when(h == nh - 1)
    def _():
        r = out_ref[...]
        out_ref[...] = r * jax.lax.rsqrt(
            jnp.mean(r * r, axis=-1, keepdims=True) + EPS)


def _ffn_sorted(xs, texp, w1_0, w3_0, w2_0, w1_1, w3_1, w2_1):
    nps, d = xs.shape
    hidden = w1_0.shape[0]
    nt = nps // TILE
    hc = hidden // HCHUNK

    def _w1map(expert):
        return lambda t, h, s: (jnp.where(s[t] == expert, h, 0), 0)

    def _w2map(expert):
        return lambda t, h, s: (0, jnp.where(s[t] == expert, h, 0))

    grid_spec = pltpu.PrefetchScalarGridSpec(
        num_scalar_prefetch=1,
        grid=(nt, hc),
        in_specs=[
            pl.BlockSpec((TILE, d), lambda t, h, s: (t, 0)),
            pl.BlockSpec((HCHUNK, d), _w1map(0)),
            pl.BlockSpec((HCHUNK, d), _w1map(0)),
            pl.BlockSpec((d, HCHUNK), _w2map(0)),
            pl.BlockSpec((HCHUNK, d), _w1map(1)),
            pl.BlockSpec((HCHUNK, d), _w1map(1)),
            pl.BlockSpec((d, HCHUNK), _w2map(1)),
        ],
        out_specs=pl.BlockSpec((TILE, d), lambda t, h, s: (t, 0)),
    )
    return pl.pallas_call(
        _ffn_body,
        grid_spec=grid_spec,
        out_shape=jax.ShapeDtypeStruct((nps, d), jnp.float32),
        compiler_params=pltpu.CompilerParams(
            dimension_semantics=("parallel", "arbitrary")),
    )(texp, xs, w1_0, w3_0, w2_0, w1_1, w3_1, w2_1)


def kernel(x, modality_masks, w1_0, w3_0, w2_0, w1_1, w3_1, w2_1):
    b, s, d = x.shape
    n = b * s
    nps = n + 2 * TILE

    ids = modality_masks[1].reshape(n).astype(jnp.int32)
    c1 = jnp.cumsum(ids)
    n0 = n - c1[-1]
    n0p = ((n0 + TILE - 1) // TILE) * TILE
    iot = jnp.arange(n, dtype=jnp.int32)
    dst = jnp.where(ids == 0, iot - c1, n0p + c1 - 1).astype(jnp.int32)
    src = jnp.zeros((nps,), jnp.int32).at[dst].set(iot)
    texp = (jnp.arange(nps // TILE, dtype=jnp.int32) >= (n0p // TILE)
            ).astype(jnp.int32)

    bf = jnp.bfloat16
    xf = x.reshape(n, d)
    xs = _sc_gather(xf, src)
    ys = _ffn_sorted(xs, texp,
                     w1_0.astype(bf), w3_0.astype(bf), w2_0.astype(bf),
                     w1_1.astype(bf), w3_1.astype(bf), w2_1.astype(bf))
    out = _sc_gather(ys, dst)
    return out.reshape(b, s, d)


# speedup vs baseline: 1.8458x; 1.0872x over previous
import functools

import jax
import jax.numpy as jnp
from jax.experimental import pallas as pl
from jax.experimental.pallas import tpu as pltpu
from jax.experimental.pallas import tpu_sc as plsc

EPS = 1e-5
TILE = 512
HCHUNK = 512
SC_NW = 32
SC_CH = 32


def _sc_gather(table, idx):
    n_idx = idx.shape[0]
    d = table.shape[1]
    b_per_w = n_idx // SC_NW
    assert n_idx % (8 * SC_NW) == 0
    assert b_per_w % SC_CH == 0
    mesh = plsc.VectorSubcoreMesh(core_axis_name="c", subcore_axis_name="s")

    @functools.partial(
        pl.kernel,
        mesh=mesh,
        out_type=jax.ShapeDtypeStruct((n_idx, d), table.dtype),
        scratch_types=[
            pltpu.VMEM((SC_CH,), jnp.int32),
            pltpu.VMEM((SC_CH, d), table.dtype),
            pltpu.SemaphoreType.DMA,
        ],
    )
    def gk(table_hbm, idx_hbm, out_hbm, idx_v, rows_v, sem):
        wid = jax.lax.axis_index("s") * 2 + jax.lax.axis_index("c")
        base = wid * b_per_w

        @pl.loop(0, b_per_w // SC_CH)
        def _(c):
            off = base + c * SC_CH
            pltpu.sync_copy(idx_hbm.at[pl.ds(off, SC_CH)], idx_v)
            pltpu.async_copy(table_hbm.at[idx_v], rows_v, sem).wait()
            pltpu.sync_copy(rows_v, out_hbm.at[pl.ds(off, SC_CH)])

    return gk(table, idx)


def _ffn_body(texp_ref, x_ref, w1_ref, w3_ref, w2_ref, out_ref):
    h = pl.program_id(1)
    nh = pl.num_programs(1)
    xb = x_ref[...].astype(jnp.bfloat16)
    w1 = w1_ref[0]
    w3 = w3_ref[0]
    w2 = w2_ref[0]
    dn = (((1,), (1,)), ((), ()))
    h1 = jax.lax.dot_general(xb, w1, dn, preferred_element_type=jnp.float32)
    h3 = jax.lax.dot_general(xb, w3, dn, preferred_element_type=jnp.float32)
    g = (h1 * jax.nn.sigmoid(h1) * h3).astype(jnp.bfloat16)
    contrib = jax.lax.dot_general(g, w2, dn, preferred_element_type=jnp.float32)

    @pl.when(h == 0)
    def _():
        out_ref[...] = contrib

    @pl.when(h > 0)
    def _():
        out_ref[...] += contrib

    @pl.when(h == nh - 1)
    def _():
        r = out_ref[...]
        out_ref[...] = r * jax.lax.rsqrt(
            jnp.mean(r * r, axis=-1, keepdims=True) + EPS)


def _ffn_sorted(xs, texp, w1s, w3s, w2s):
    nps, d = xs.shape
    hidden = w1s.shape[1]
    nt = nps // TILE
    hc = hidden // HCHUNK

    grid_spec = pltpu.PrefetchScalarGridSpec(
        num_scalar_prefetch=1,
        grid=(nt, hc),
        in_specs=[
            pl.BlockSpec((TILE, d), lambda t, h, s: (t, 0)),
            pl.BlockSpec((1, HCHUNK, d), lambda t, h, s: (s[t], h, 0)),
            pl.BlockSpec((1, HCHUNK, d), lambda t, h, s: (s[t], h, 0)),
            pl.BlockSpec((1, d, HCHUNK), lambda t, h, s: (s[t], 0, h)),
        ],
        out_specs=pl.BlockSpec((TILE, d), lambda t, h, s: (t, 0)),
    )
    return pl.pallas_call(
        _ffn_body,
        grid_spec=grid_spec,
        out_shape=jax.ShapeDtypeStruct((nps, d), jnp.float32),
        compiler_params=pltpu.CompilerParams(
            dimension_semantics=("parallel", "arbitrary")),
    )(texp, xs, w1s, w3s, w2s)


def kernel(x, modality_masks, w1_0, w3_0, w2_0, w1_1, w3_1, w2_1):
    b, s, d = x.shape
    n = b * s
    nps = n + 2 * TILE

    ids = modality_masks[1].reshape(n).astype(jnp.int32)
    c1 = jnp.cumsum(ids)
    n0 = n - c1[-1]
    n0p = ((n0 + TILE - 1) // TILE) * TILE
    iot = jnp.arange(n, dtype=jnp.int32)
    dst = jnp.where(ids == 0, iot - c1, n0p + c1 - 1).astype(jnp.int32)
    src = jnp.zeros((nps,), jnp.int32).at[dst].set(iot)
    texp = (jnp.arange(nps // TILE, dtype=jnp.int32) >= (n0p // TILE)
            ).astype(jnp.int32)

    bf = jnp.bfloat16
    xf = x.reshape(n, d)
    xs = _sc_gather(xf, src)
    w1s = jnp.stack([w1_0, w1_1]).astype(bf)
    w3s = jnp.stack([w3_0, w3_1]).astype(bf)
    w2s = jnp.stack([w2_0, w2_1]).astype(bf)
    ys = _ffn_sorted(xs, texp, w1s, w3s, w2s)
    out = _sc_gather(ys, dst)
    return out.reshape(b, s, d)


# speedup vs baseline: 1.8475x; 1.0009x over previous
import functools

import jax
import jax.numpy as jnp
from jax.experimental import pallas as pl
from jax.experimental.pallas import tpu as pltpu
from jax.experimental.pallas import tpu_sc as plsc

EPS = 1e-5
TILE = 512
HCHUNK = 512
SC_NW = 32
SC_CH = 16


def _sc_gather(table, idx):
    n_idx = idx.shape[0]
    d = table.shape[1]
    b_per_w = n_idx // SC_NW
    nch = b_per_w // SC_CH
    assert n_idx % (8 * SC_NW) == 0
    assert b_per_w % (2 * SC_CH) == 0
    mesh = plsc.VectorSubcoreMesh(core_axis_name="c", subcore_axis_name="s")

    @functools.partial(
        pl.kernel,
        mesh=mesh,
        out_type=jax.ShapeDtypeStruct((n_idx, d), table.dtype),
        scratch_types=[
            pltpu.VMEM((b_per_w,), jnp.int32),
            pltpu.VMEM((SC_CH, d), table.dtype),
            pltpu.VMEM((SC_CH, d), table.dtype),
            pltpu.SemaphoreType.DMA,
            pltpu.SemaphoreType.DMA,
            pltpu.SemaphoreType.DMA,
            pltpu.SemaphoreType.DMA,
        ],
    )
    def gk(table_hbm, idx_hbm, out_hbm, idx_v, rows0, rows1, gs0, gs1,
           ws0, ws1):
        wid = jax.lax.axis_index("s") * 2 + jax.lax.axis_index("c")
        base = wid * b_per_w
        pltpu.sync_copy(idx_hbm.at[pl.ds(base, b_per_w)], idx_v)

        @pl.loop(0, nch // 2)
        def _(p):
            c0 = 2 * p * SC_CH
            c1 = c0 + SC_CH
            g0 = pltpu.async_copy(
                table_hbm.at[idx_v.at[pl.ds(c0, SC_CH)]], rows0, gs0)
            g1 = pltpu.async_copy(
                table_hbm.at[idx_v.at[pl.ds(c1, SC_CH)]], rows1, gs1)
            g0.wait()
            w0 = pltpu.async_copy(rows0, out_hbm.at[pl.ds(base + c0, SC_CH)],
                                  ws0)
            g1.wait()
            w1 = pltpu.async_copy(rows1, out_hbm.at[pl.ds(base + c1, SC_CH)],
                                  ws1)
            w0.wait()
            w1.wait()

    return gk(table, idx)


def _ffn_body(texp_ref, x_ref, w1_ref, w3_ref, w2_ref, out_ref):
    h = pl.program_id(1)
    nh = pl.num_programs(1)
    xb = x_ref[...].astype(jnp.bfloat16)
    w1 = w1_ref[0]
    w3 = w3_ref[0]
    w2 = w2_ref[0]
    dn = (((1,), (1,)), ((), ()))
    h1 = jax.lax.dot_general(xb, w1, dn, preferred_element_type=jnp.float32)
    h3 = jax.lax.dot_general(xb, w3, dn, preferred_element_type=jnp.float32)
    g = (h1 * jax.nn.sigmoid(h1) * h3).astype(jnp.bfloat16)
    contrib = jax.lax.dot_general(g, w2, dn, preferred_element_type=jnp.float32)

    @pl.when(h == 0)
    def _():
        out_ref[...] = contrib

    @pl.when(h > 0)
    def _():
        out_ref[...] += contrib

    @pl.when(h == nh - 1)
    def _():
        r = out_ref[...]
        out_ref[...] = r * jax.lax.rsqrt(
            jnp.mean(r * r, axis=-1, keepdims=True) + EPS)


def _ffn_sorted(xs, texp, w1s, w3s, w2s):
    nps, d = xs.shape
    hidden = w1s.shape[1]
    nt = nps // TILE
    hc = hidden // HCHUNK

    grid_spec = pltpu.PrefetchScalarGridSpec(
        num_scalar_prefetch=1,
        grid=(nt, hc),
        in_specs=[
            pl.BlockSpec((TILE, d), lambda t, h, s: (t, 0)),
            pl.BlockSpec((1, HCHUNK, d), lambda t, h, s: (s[t], h, 0)),
            pl.BlockSpec((1, HCHUNK, d), lambda t, h, s: (s[t], h, 0)),
            pl.BlockSpec((1, d, HCHUNK), lambda t, h, s: (s[t], 0, h)),
        ],
        out_specs=pl.BlockSpec((TILE, d), lambda t, h, s: (t, 0)),
    )
    return pl.pallas_call(
        _ffn_body,
        grid_spec=grid_spec,
        out_shape=jax.ShapeDtypeStruct((nps, d), jnp.float32),
        compiler_params=pltpu.CompilerParams(
            dimension_semantics=("parallel", "arbitrary")),
    )(texp, xs, w1s, w3s, w2s)


def kernel(x, modality_masks, w1_0, w3_0, w2_0, w1_1, w3_1, w2_1):
    b, s, d = x.shape
    n = b * s
    nps = n + 2 * TILE

    ids = modality_masks[1].reshape(n).astype(jnp.int32)
    c1 = jnp.cumsum(ids)
    n0 = n - c1[-1]
    n0p = ((n0 + TILE - 1) // TILE) * TILE
    iot = jnp.arange(n, dtype=jnp.int32)
    dst = jnp.where(ids == 0, iot - c1, n0p + c1 - 1).astype(jnp.int32)
    src = jnp.zeros((nps,), jnp.int32).at[dst].set(iot)
    texp = (jnp.arange(nps // TILE, dtype=jnp.int32) >= (n0p // TILE)
            ).astype(jnp.int32)

    bf = jnp.bfloat16
    xf = x.reshape(n, d)
    xs = _sc_gather(xf, src)
    w1s = jnp.stack([w1_0, w1_1]).astype(bf)
    w3s = jnp.stack([w3_0, w3_1]).astype(bf)
    w2s = jnp.stack([w2_0, w2_1]).astype(bf)
    ys = _ffn_sorted(xs, texp, w1s, w3s, w2s)
    out = _sc_gather(ys, dst)
    return out.reshape(b, s, d)


# speedup vs baseline: 1.9544x; 1.0579x over previous
import functools

import jax
import jax.numpy as jnp
from jax.experimental import pallas as pl
from jax.experimental.pallas import tpu as pltpu
from jax.experimental.pallas import tpu_sc as plsc

EPS = 1e-5
TILE = 512
HCHUNK = 512
SC_NW = 32
SC_CH = 16


def _sc_gather(table, idx):
    n_idx = idx.shape[0]
    d = table.shape[1]
    b_per_w = n_idx // SC_NW
    nch = b_per_w // SC_CH
    assert n_idx % (8 * SC_NW) == 0
    assert b_per_w % (2 * SC_CH) == 0
    mesh = plsc.VectorSubcoreMesh(core_axis_name="c", subcore_axis_name="s")

    @functools.partial(
        pl.kernel,
        mesh=mesh,
        out_type=jax.ShapeDtypeStruct((n_idx, d), table.dtype),
        scratch_types=[
            pltpu.VMEM((b_per_w,), jnp.int32),
            pltpu.VMEM((SC_CH, d), table.dtype),
            pltpu.VMEM((SC_CH, d), table.dtype),
            pltpu.SemaphoreType.DMA,
            pltpu.SemaphoreType.DMA,
            pltpu.SemaphoreType.DMA,
            pltpu.SemaphoreType.DMA,
        ],
    )
    def gk(table_hbm, idx_hbm, out_hbm, idx_v, rows0, rows1, gs0, gs1,
           ws0, ws1):
        wid = jax.lax.axis_index("s") * 2 + jax.lax.axis_index("c")
        base = wid * b_per_w
        pltpu.sync_copy(idx_hbm.at[pl.ds(base, b_per_w)], idx_v)

        @pl.loop(0, nch // 2)
        def _(p):
            c0 = 2 * p * SC_CH
            c1 = c0 + SC_CH
            g0 = pltpu.async_copy(
                table_hbm.at[idx_v.at[pl.ds(c0, SC_CH)]], rows0, gs0)
            g1 = pltpu.async_copy(
                table_hbm.at[idx_v.at[pl.ds(c1, SC_CH)]], rows1, gs1)
            g0.wait()
            w0 = pltpu.async_copy(rows0, out_hbm.at[pl.ds(base + c0, SC_CH)],
                                  ws0)
            g1.wait()
            w1 = pltpu.async_copy(rows1, out_hbm.at[pl.ds(base + c1, SC_CH)],
                                  ws1)
            w0.wait()
            w1.wait()

    return gk(table, idx)


def _sc_scatter(rows, idx, out_rows):
    n_idx = idx.shape[0]
    d = rows.shape[1]
    b_per_w = n_idx // SC_NW
    nch = b_per_w // SC_CH
    assert n_idx % (8 * SC_NW) == 0
    assert b_per_w % (2 * SC_CH) == 0
    mesh = plsc.VectorSubcoreMesh(core_axis_name="c", subcore_axis_name="s")

    @functools.partial(
        pl.kernel,
        mesh=mesh,
        out_type=jax.ShapeDtypeStruct((out_rows, d), rows.dtype),
        scratch_types=[
            pltpu.VMEM((b_per_w,), jnp.int32),
            pltpu.VMEM((SC_CH, d), rows.dtype),
            pltpu.VMEM((SC_CH, d), rows.dtype),
            pltpu.SemaphoreType.DMA,
            pltpu.SemaphoreType.DMA,
            pltpu.SemaphoreType.DMA,
            pltpu.SemaphoreType.DMA,
        ],
    )
    def sk(rows_hbm, idx_hbm, out_hbm, idx_v, rows0, rows1, gs0, gs1,
           ws0, ws1):
        wid = jax.lax.axis_index("s") * 2 + jax.lax.axis_index("c")
        base = wid * b_per_w
        pltpu.sync_copy(idx_hbm.at[pl.ds(base, b_per_w)], idx_v)

        @pl.loop(0, nch // 2)
        def _(p):
            c0 = 2 * p * SC_CH
            c1 = c0 + SC_CH
            g0 = pltpu.async_copy(rows_hbm.at[pl.ds(base + c0, SC_CH)],
                                  rows0, gs0)
            g1 = pltpu.async_copy(rows_hbm.at[pl.ds(base + c1, SC_CH)],
                                  rows1, gs1)
            g0.wait()
            w0 = pltpu.async_copy(rows0, out_hbm.at[idx_v.at[pl.ds(c0, SC_CH)]],
                                  ws0)
            g1.wait()
            w1 = pltpu.async_copy(rows1, out_hbm.at[idx_v.at[pl.ds(c1, SC_CH)]],
                                  ws1)
            w0.wait()
            w1.wait()

    return sk(rows, idx)


def _ffn_body(texp_ref, x_ref, w1_ref, w3_ref, w2_ref, out_ref):
    h = pl.program_id(1)
    nh = pl.num_programs(1)
    xb = x_ref[...].astype(jnp.bfloat16)
    w1 = w1_ref[0]
    w3 = w3_ref[0]
    w2 = w2_ref[0]
    dn = (((1,), (1,)), ((), ()))
    h1 = jax.lax.dot_general(xb, w1, dn, preferred_element_type=jnp.float32)
    h3 = jax.lax.dot_general(xb, w3, dn, preferred_element_type=jnp.float32)
    g = (h1 * jax.nn.sigmoid(h1) * h3).astype(jnp.bfloat16)
    contrib = jax.lax.dot_general(g, w2, dn, preferred_element_type=jnp.float32)

    @pl.when(h == 0)
    def _():
        out_ref[...] = contrib

    @pl.when(h > 0)
    def _():
        out_ref[...] += contrib

    @pl.when(h == nh - 1)
    def _():
        r = out_ref[...]
        out_ref[...] = r * jax.lax.rsqrt(
            jnp.mean(r * r, axis=-1, keepdims=True) + EPS)


def _ffn_sorted(xs, texp, w1s, w3s, w2s):
    nps, d = xs.shape
    hidden = w1s.shape[1]
    nt = nps // TILE
    hc = hidden // HCHUNK

    grid_spec = pltpu.PrefetchScalarGridSpec(
        num_scalar_prefetch=1,
        grid=(nt, hc),
        in_specs=[
            pl.BlockSpec((TILE, d), lambda t, h, s: (t, 0)),
            pl.BlockSpec((1, HCHUNK, d), lambda t, h, s: (s[t], h, 0)),
            pl.BlockSpec((1, HCHUNK, d), lambda t, h, s: (s[t], h, 0)),
            pl.BlockSpec((1, d, HCHUNK), lambda t, h, s: (s[t], 0, h)),
        ],
        out_specs=pl.BlockSpec((TILE, d), lambda t, h, s: (t, 0)),
    )
    return pl.pallas_call(
        _ffn_body,
        grid_spec=grid_spec,
        out_shape=jax.ShapeDtypeStruct((nps, d), jnp.float32),
        compiler_params=pltpu.CompilerParams(
            dimension_semantics=("parallel", "arbitrary")),
    )(texp, xs, w1s, w3s, w2s)


def kernel(x, modality_masks, w1_0, w3_0, w2_0, w1_1, w3_1, w2_1):
    b, s, d = x.shape
    n = b * s
    nps = n + 2 * TILE

    ids = modality_masks[1].reshape(n).astype(jnp.int32)
    c1 = jnp.cumsum(ids)
    n0 = n - c1[-1]
    n0p = ((n0 + TILE - 1) // TILE) * TILE
    iot = jnp.arange(n, dtype=jnp.int32)
    dst = jnp.where(ids == 0, iot - c1, n0p + c1 - 1).astype(jnp.int32)
    texp = (jnp.arange(nps // TILE, dtype=jnp.int32) >= (n0p // TILE)
            ).astype(jnp.int32)

    bf = jnp.bfloat16
    xf = x.reshape(n, d)
    xs = _sc_scatter(xf, dst, nps)
    w1s = jnp.stack([w1_0, w1_1]).astype(bf)
    w3s = jnp.stack([w3_0, w3_1]).astype(bf)
    w2s = jnp.stack([w2_0, w2_1]).astype(bf)
    ys = _ffn_sorted(xs, texp, w1s, w3s, w2s)
    out = _sc_gather(ys, dst)
    return out.reshape(b, s, d)


# speedup vs baseline: 1.9668x; 1.0063x over previous
import functools

import jax
import jax.numpy as jnp
from jax.experimental import pallas as pl
from jax.experimental.pallas import tpu as pltpu
from jax.experimental.pallas import tpu_sc as plsc

EPS = 1e-5
TILE = 512
HCHUNK = 512
SC_NW = 32
SC_CH = 16


def _sc_gather(table, idx):
    n_idx = idx.shape[0]
    d = table.shape[1]
    b_per_w = n_idx // SC_NW
    nch = b_per_w // SC_CH
    assert n_idx % (8 * SC_NW) == 0
    assert b_per_w % (2 * SC_CH) == 0
    mesh = plsc.VectorSubcoreMesh(core_axis_name="c", subcore_axis_name="s")

    @functools.partial(
        pl.kernel,
        mesh=mesh,
        out_type=jax.ShapeDtypeStruct((n_idx, d), table.dtype),
        scratch_types=[
            pltpu.VMEM((b_per_w,), jnp.int32),
            pltpu.VMEM((SC_CH, d), table.dtype),
            pltpu.VMEM((SC_CH, d), table.dtype),
            pltpu.SemaphoreType.DMA,
            pltpu.SemaphoreType.DMA,
            pltpu.SemaphoreType.DMA,
            pltpu.SemaphoreType.DMA,
        ],
    )
    def gk(table_hbm, idx_hbm, out_hbm, idx_v, rows0, rows1, gs0, gs1,
           ws0, ws1):
        wid = jax.lax.axis_index("s") * 2 + jax.lax.axis_index("c")
        base = wid * b_per_w
        pltpu.sync_copy(idx_hbm.at[pl.ds(base, b_per_w)], idx_v)

        @pl.loop(0, nch // 2)
        def _(p):
            c0 = 2 * p * SC_CH
            c1 = c0 + SC_CH
            g0 = pltpu.async_copy(
                table_hbm.at[idx_v.at[pl.ds(c0, SC_CH)]], rows0, gs0)
            g1 = pltpu.async_copy(
                table_hbm.at[idx_v.at[pl.ds(c1, SC_CH)]], rows1, gs1)
            g0.wait()
            w0 = pltpu.async_copy(rows0, out_hbm.at[pl.ds(base + c0, SC_CH)],
                                  ws0)
            g1.wait()
            w1 = pltpu.async_copy(rows1, out_hbm.at[pl.ds(base + c1, SC_CH)],
                                  ws1)
            w0.wait()
            w1.wait()

    return gk(table, idx)


def _sc_scatter(rows, idx, out_rows):
    n_idx = idx.shape[0]
    d = rows.shape[1]
    b_per_w = n_idx // SC_NW
    nch = b_per_w // SC_CH
    assert n_idx % (8 * SC_NW) == 0
    assert b_per_w % (2 * SC_CH) == 0
    mesh = plsc.VectorSubcoreMesh(core_axis_name="c", subcore_axis_name="s")

    @functools.partial(
        pl.kernel,
        mesh=mesh,
        out_type=jax.ShapeDtypeStruct((out_rows, d), rows.dtype),
        scratch_types=[
            pltpu.VMEM((b_per_w,), jnp.int32),
            pltpu.VMEM((SC_CH, d), rows.dtype),
            pltpu.VMEM((SC_CH, d), rows.dtype),
            pltpu.SemaphoreType.DMA,
            pltpu.SemaphoreType.DMA,
            pltpu.SemaphoreType.DMA,
            pltpu.SemaphoreType.DMA,
        ],
    )
    def sk(rows_hbm, idx_hbm, out_hbm, idx_v, rows0, rows1, gs0, gs1,
           ws0, ws1):
        wid = jax.lax.axis_index("s") * 2 + jax.lax.axis_index("c")
        base = wid * b_per_w
        pltpu.sync_copy(idx_hbm.at[pl.ds(base, b_per_w)], idx_v)

        @pl.loop(0, nch // 2)
        def _(p):
            c0 = 2 * p * SC_CH
            c1 = c0 + SC_CH
            g0 = pltpu.async_copy(rows_hbm.at[pl.ds(base + c0, SC_CH)],
                                  rows0, gs0)
            g1 = pltpu.async_copy(rows_hbm.at[pl.ds(base + c1, SC_CH)],
                                  rows1, gs1)
            g0.wait()
            w0 = pltpu.async_copy(rows0, out_hbm.at[idx_v.at[pl.ds(c0, SC_CH)]],
                                  ws0)
            g1.wait()
            w1 = pltpu.async_copy(rows1, out_hbm.at[idx_v.at[pl.ds(c1, SC_CH)]],
                                  ws1)
            w0.wait()
            w1.wait()

    return sk(rows, idx)


def _ffn_body(texp_ref, x_ref, w1_ref, w3_ref, w2_ref, out_ref):
    h = pl.program_id(1)
    nh = pl.num_programs(1)
    xb = x_ref[...].astype(jnp.bfloat16)
    w1 = w1_ref[0]
    w3 = w3_ref[0]
    w2 = w2_ref[0]
    dn = (((1,), (0,)), ((), ()))
    h1 = jax.lax.dot_general(xb, w1, dn, preferred_element_type=jnp.float32)
    h3 = jax.lax.dot_general(xb, w3, dn, preferred_element_type=jnp.float32)
    g = (h1 * jax.nn.sigmoid(h1) * h3).astype(jnp.bfloat16)
    contrib = jax.lax.dot_general(g, w2, dn, preferred_element_type=jnp.float32)

    @pl.when(h == 0)
    def _():
        out_ref[...] = contrib

    @pl.when(h > 0)
    def _():
        out_ref[...] += contrib

    @pl.when(h == nh - 1)
    def _():
        r = out_ref[...]
        out_ref[...] = r * jax.lax.rsqrt(
            jnp.mean(r * r, axis=-1, keepdims=True) + EPS)


def _ffn_sorted(xs, texp, w1s, w3s, w2s):
    nps, d = xs.shape
    hidden = w1s.shape[2]
    nt = nps // TILE
    hc = hidden // HCHUNK

    grid_spec = pltpu.PrefetchScalarGridSpec(
        num_scalar_prefetch=1,
        grid=(nt, hc),
        in_specs=[
            pl.BlockSpec((TILE, d), lambda t, h, s: (t, 0)),
            pl.BlockSpec((1, d, HCHUNK), lambda t, h, s: (s[t], 0, h)),
            pl.BlockSpec((1, d, HCHUNK), lambda t, h, s: (s[t], 0, h)),
            pl.BlockSpec((1, HCHUNK, d), lambda t, h, s: (s[t], h, 0)),
        ],
        out_specs=pl.BlockSpec((TILE, d), lambda t, h, s: (t, 0)),
    )
    return pl.pallas_call(
        _ffn_body,
        grid_spec=grid_spec,
        out_shape=jax.ShapeDtypeStruct((nps, d), jnp.float32),
        compiler_params=pltpu.CompilerParams(
            dimension_semantics=("parallel", "arbitrary")),
    )(texp, xs, w1s, w3s, w2s)


def kernel(x, modality_masks, w1_0, w3_0, w2_0, w1_1, w3_1, w2_1):
    b, s, d = x.shape
    n = b * s
    nps = n + 2 * TILE

    ids = modality_masks[1].reshape(n).astype(jnp.int32)
    c1 = jnp.cumsum(ids)
    n0 = n - c1[-1]
    n0p = ((n0 + TILE - 1) // TILE) * TILE
    iot = jnp.arange(n, dtype=jnp.int32)
    dst = jnp.where(ids == 0, iot - c1, n0p + c1 - 1).astype(jnp.int32)
    texp = (jnp.arange(nps // TILE, dtype=jnp.int32) >= (n0p // TILE)
            ).astype(jnp.int32)

    bf = jnp.bfloat16
    xf = x.reshape(n, d)
    xs = _sc_scatter(xf, dst, nps)
    w1s = jnp.stack([w1_0.T, w1_1.T]).astype(bf)
    w3s = jnp.stack([w3_0.T, w3_1.T]).astype(bf)
    w2s = jnp.stack([w2_0.T, w2_1.T]).astype(bf)
    ys = _ffn_sorted(xs, texp, w1s, w3s, w2s)
    out = _sc_gather(ys, dst)
    return out.reshape(b, s, d)


# speedup vs baseline: 2.0462x; 1.0404x over previous
import functools

import jax
import jax.numpy as jnp
from jax.experimental import pallas as pl
from jax.experimental.pallas import tpu as pltpu
from jax.experimental.pallas import tpu_sc as plsc

EPS = 1e-5
TILE = 512
HCHUNK = 1408
SC_NW = 32
SC_CH = 16


def _sc_gather(table, idx):
    n_idx = idx.shape[0]
    d = table.shape[1]
    b_per_w = n_idx // SC_NW
    nch = b_per_w // SC_CH
    assert n_idx % (8 * SC_NW) == 0
    assert b_per_w % (2 * SC_CH) == 0
    mesh = plsc.VectorSubcoreMesh(core_axis_name="c", subcore_axis_name="s")

    @functools.partial(
        pl.kernel,
        mesh=mesh,
        out_type=jax.ShapeDtypeStruct((n_idx, d), table.dtype),
        scratch_types=[
            pltpu.VMEM((b_per_w,), jnp.int32),
            pltpu.VMEM((SC_CH, d), table.dtype),
            pltpu.VMEM((SC_CH, d), table.dtype),
            pltpu.SemaphoreType.DMA,
            pltpu.SemaphoreType.DMA,
            pltpu.SemaphoreType.DMA,
            pltpu.SemaphoreType.DMA,
        ],
    )
    def gk(table_hbm, idx_hbm, out_hbm, idx_v, rows0, rows1, gs0, gs1,
           ws0, ws1):
        wid = jax.lax.axis_index("s") * 2 + jax.lax.axis_index("c")
        base = wid * b_per_w
        pltpu.sync_copy(idx_hbm.at[pl.ds(base, b_per_w)], idx_v)

        @pl.loop(0, nch // 2)
        def _(p):
            c0 = 2 * p * SC_CH
            c1 = c0 + SC_CH
            g0 = pltpu.async_copy(
                table_hbm.at[idx_v.at[pl.ds(c0, SC_CH)]], rows0, gs0)
            g1 = pltpu.async_copy(
                table_hbm.at[idx_v.at[pl.ds(c1, SC_CH)]], rows1, gs1)
            g0.wait()
            w0 = pltpu.async_copy(rows0, out_hbm.at[pl.ds(base + c0, SC_CH)],
                                  ws0)
            g1.wait()
            w1 = pltpu.async_copy(rows1, out_hbm.at[pl.ds(base + c1, SC_CH)],
                                  ws1)
            w0.wait()
            w1.wait()

    return gk(table, idx)


def _sc_scatter(rows, idx, out_rows):
    n_idx = idx.shape[0]
    d = rows.shape[1]
    b_per_w = n_idx // SC_NW
    nch = b_per_w // SC_CH
    assert n_idx % (8 * SC_NW) == 0
    assert b_per_w % (2 * SC_CH) == 0
    mesh = plsc.VectorSubcoreMesh(core_axis_name="c", subcore_axis_name="s")

    @functools.partial(
        pl.kernel,
        mesh=mesh,
        out_type=jax.ShapeDtypeStruct((out_rows, d), rows.dtype),
        scratch_types=[
            pltpu.VMEM((b_per_w,), jnp.int32),
            pltpu.VMEM((SC_CH, d), rows.dtype),
            pltpu.VMEM((SC_CH, d), rows.dtype),
            pltpu.SemaphoreType.DMA,
            pltpu.SemaphoreType.DMA,
            pltpu.SemaphoreType.DMA,
            pltpu.SemaphoreType.DMA,
        ],
    )
    def sk(rows_hbm, idx_hbm, out_hbm, idx_v, rows0, rows1, gs0, gs1,
           ws0, ws1):
        wid = jax.lax.axis_index("s") * 2 + jax.lax.axis_index("c")
        base = wid * b_per_w
        pltpu.sync_copy(idx_hbm.at[pl.ds(base, b_per_w)], idx_v)

        @pl.loop(0, nch // 2)
        def _(p):
            c0 = 2 * p * SC_CH
            c1 = c0 + SC_CH
            g0 = pltpu.async_copy(rows_hbm.at[pl.ds(base + c0, SC_CH)],
                                  rows0, gs0)
            g1 = pltpu.async_copy(rows_hbm.at[pl.ds(base + c1, SC_CH)],
                                  rows1, gs1)
            g0.wait()
            w0 = pltpu.async_copy(rows0, out_hbm.at[idx_v.at[pl.ds(c0, SC_CH)]],
                                  ws0)
            g1.wait()
            w1 = pltpu.async_copy(rows1, out_hbm.at[idx_v.at[pl.ds(c1, SC_CH)]],
                                  ws1)
            w0.wait()
            w1.wait()

    return sk(rows, idx)


def _ffn_body(texp_ref, x_ref, w1_ref, w3_ref, w2_ref, out_ref):
    h = pl.program_id(1)
    nh = pl.num_programs(1)
    xb = x_ref[...].astype(jnp.bfloat16)
    w1 = w1_ref[0]
    w3 = w3_ref[0]
    w2 = w2_ref[0]
    dn = (((1,), (0,)), ((), ()))
    h1 = jax.lax.dot_general(xb, w1, dn, preferred_element_type=jnp.float32)
    h3 = jax.lax.dot_general(xb, w3, dn, preferred_element_type=jnp.float32)
    g = (h1 * jax.nn.sigmoid(h1) * h3).astype(jnp.bfloat16)
    contrib = jax.lax.dot_general(g, w2, dn, preferred_element_type=jnp.float32)

    @pl.when(h == 0)
    def _():
        out_ref[...] = contrib

    @pl.when(h > 0)
    def _():
        out_ref[...] += contrib

    @pl.when(h == nh - 1)
    def _():
        r = out_ref[...]
        out_ref[...] = r * jax.lax.rsqrt(
            jnp.mean(r * r, axis=-1, keepdims=True) + EPS)


def _ffn_sorted(xs, texp, w1s, w3s, w2s):
    nps, d = xs.shape
    hidden = w1s.shape[2]
    nt = nps // TILE
    hc = hidden // HCHUNK

    grid_spec = pltpu.PrefetchScalarGridSpec(
        num_scalar_prefetch=1,
        grid=(nt, hc),
        in_specs=[
            pl.BlockSpec((TILE, d), lambda t, h, s: (t, 0)),
            pl.BlockSpec((1, d, HCHUNK), lambda t, h, s: (s[t], 0, h)),
            pl.BlockSpec((1, d, HCHUNK), lambda t, h, s: (s[t], 0, h)),
            pl.BlockSpec((1, HCHUNK, d), lambda t, h, s: (s[t], h, 0)),
        ],
        out_specs=pl.BlockSpec((TILE, d), lambda t, h, s: (t, 0)),
    )
    return pl.pallas_call(
        _ffn_body,
        grid_spec=grid_spec,
        out_shape=jax.ShapeDtypeStruct((nps, d), jnp.float32),
        compiler_params=pltpu.CompilerParams(
            dimension_semantics=("parallel", "arbitrary")),
    )(texp, xs, w1s, w3s, w2s)


def kernel(x, modality_masks, w1_0, w3_0, w2_0, w1_1, w3_1, w2_1):
    b, s, d = x.shape
    n = b * s
    nps = n + 2 * TILE

    ids = modality_masks[1].reshape(n).astype(jnp.int32)
    c1 = jnp.cumsum(ids)
    n0 = n - c1[-1]
    n0p = ((n0 + TILE - 1) // TILE) * TILE
    iot = jnp.arange(n, dtype=jnp.int32)
    dst = jnp.where(ids == 0, iot - c1, n0p + c1 - 1).astype(jnp.int32)
    texp = (jnp.arange(nps // TILE, dtype=jnp.int32) >= (n0p // TILE)
            ).astype(jnp.int32)

    bf = jnp.bfloat16
    xf = x.reshape(n, d)
    xs = _sc_scatter(xf, dst, nps)
    w1s = jnp.stack([w1_0.T, w1_1.T]).astype(bf)
    w3s = jnp.stack([w3_0.T, w3_1.T]).astype(bf)
    w2s = jnp.stack([w2_0.T, w2_1.T]).astype(bf)
    ys = _ffn_sorted(xs, texp, w1s, w3s, w2s)
    out = _sc_gather(ys, dst)
    return out.reshape(b, s, d)
